# baseline (speedup 1.0000x reference)
import functools

import numpy as np

import jax
import jax.numpy as jnp
from jax.experimental import pallas as pl
from jax.experimental.pallas import tpu as pltpu
from jax.experimental.pallas import tpu_sc as plsc

_D = 1024
_B = 8
_K = max(1, int(0.1 * _D * _D))
_CHUNK = 128
_C = _D // _CHUNK
_MASK31 = 0x7FFFFFFF
_TPB = (_D * _D) // 16
_HALF = _TPB // 2


def _to_key(x):
    bits = jax.lax.bitcast_convert_type(x, jnp.int32)
    return jnp.where(bits < 0, bits ^ _MASK31, bits)


def _build_body(env_ref, temp_ref, base_ref, delta_ref,
                logits_ref, soft_ref, base_vmem):
    b = pl.program_id(0)
    i = pl.program_id(1)
    row = i * _CHUNK

    @pl.when(b == 0)
    def _fill_cache():
        base_vmem[pl.ds(row, _CHUNK), :] = base_ref[...]

    x = base_vmem[pl.ds(row, _CHUNK), :] + delta_ref[0]
    logits_ref[0] = x
    soft_ref[0] = jax.nn.sigmoid(x * (1.0 / temp_ref[0]))


def _emit_body(kth_ref, logits_ref, a_ref):
    b = pl.program_id(0)
    x = logits_ref[0]
    key = _to_key(x)
    kth = kth_ref[b]
    a_ref[0] = jnp.where(key >= kth, jax.nn.sigmoid(x), 0.0)


def _sc_select(logits_flat):
    mesh = plsc.VectorSubcoreMesh(core_axis_name="c", subcore_axis_name="s")

    @functools.partial(
        pl.kernel, mesh=mesh,
        compiler_params=pltpu.CompilerParams(needs_layout_passes=False),
        out_type=jax.ShapeDtypeStruct((2, 16), jnp.int32),
        scratch_types=[
            pltpu.VMEM((_TPB,), jnp.float32),
            pltpu.VMEM((4096,), jnp.int32),
            pltpu.VMEM((256,), jnp.int32),
            pltpu.VMEM((256,), jnp.int32),
            pltpu.VMEM((256,), jnp.int32),
            pltpu.VMEM((16,), jnp.int32),
            pltpu.VMEM((256,), jnp.int32),
            pltpu.VMEM_SHARED((256,), jnp.int32),
        ],
    )
    def sel(logits_hbm, out_hbm,
            keys_vm, hist_vm, red_vm, shr_vm, zeros_vm, kth_vm, iota_vm,
            shared):
        core = jax.lax.axis_index("c")
        sid = jax.lax.axis_index("s")
        lane = jax.lax.iota(jnp.int32, 16)
        zvec = jnp.zeros((16,), jnp.int32)
        ones = jnp.ones((16,), jnp.int32)

        def _perm(v, idx):
            return jax.lax.gather(
                v, idx[:, None],
                jax.lax.GatherDimensionNumbers(
                    offset_dims=(), collapsed_slice_dims=(0,),
                    start_index_map=(0,)),
                (1,), mode=jax.lax.GatherScatterMode.PROMISE_IN_BOUNDS)

        def splat_sum(v):
            for kk in (1, 2, 4, 8):
                v = v + _perm(v, lane ^ kk)
            return v

        def csum(v):
            for kk in (1, 2, 4, 8):
                g = _perm(v, jnp.maximum(lane - kk, 0))
                v = v + jnp.where(lane >= kk, g, zvec)
            return v

        for v in range(16):
            zeros_vm[pl.ds(v * 16, 16)] = zvec
            iota_vm[pl.ds(v * 16, 16)] = lane + v * 16
        kth_vec = zvec
        off = sid * _TPB
        un = 8
        for b in range(4):
            bg = core * 4 + b
            pltpu.sync_copy(logits_hbm.at[bg, pl.ds(off, _TPB)], keys_vm)
            req = jnp.full((16,), _K, jnp.int32)
            pref = zvec
            for t in range(4):
                def _zh(v, _):
                    hist_vm[pl.ds(v * 16, 16)] = zvec
                    return 0

                jax.lax.fori_loop(0, 256, _zh, 0)
                shd = 24 - 8 * t

                def _scan(g, _, t=t, shd=shd, pref=pref):
                    for u in range(un):
                        ptr = (g * un + u) * 16
                        raw = keys_vm[pl.ds(ptr, 16)]
                        if t == 0:
                            key = _to_key(raw)
                            keys_vm[pl.ds(ptr, 16)] = (
                                jax.lax.bitcast_convert_type(key, jnp.float32))
                            dig = jax.lax.shift_right_arithmetic(key, 24) + 128
                            idx = lane * 256 + dig
                            plsc.addupdate_scatter(hist_vm, [idx], ones,
                                                   mask=lane >= 0)
                        else:
                            kv = jax.lax.bitcast_convert_type(raw, jnp.int32)
                            dig = jax.lax.shift_right_arithmetic(kv, shd) & 255
                            hi = jax.lax.shift_right_arithmetic(kv, shd + 8)
                            idx = lane * 256 + dig
                            plsc.addupdate_scatter(hist_vm, [idx], ones,
                                                   mask=hi == pref)
                    return 0

                jax.lax.fori_loop(0, _TPB // 16 // un, _scan, 0)

                def _red(blk, _):
                    acc = zvec
                    for l in range(16):
                        acc = acc + hist_vm[pl.ds(l * 256 + blk * 16, 16)]
                    red_vm[pl.ds(blk * 16, 16)] = acc
                    return 0

                jax.lax.fori_loop(0, 16, _red, 0)
                plsc.subcore_barrier()

                @pl.when(sid == 0)
                def _z():
                    pltpu.sync_copy(zeros_vm, shared)

                plsc.subcore_barrier()
                pltpu.sync_copy(red_vm, shared.at[iota_vm], add=True)
                plsc.subcore_barrier()
                pltpu.sync_copy(shared, shr_vm)
                tv = [splat_sum(shr_vm[pl.ds(v * 16, 16)]) for v in range(16)]
                abv = [None] * 16
                acc = zvec
                for v in range(15, -1, -1):
                    abv[v] = acc
                    acc = acc + tv[v]
                s_sel = zvec
                above_sel = zvec
                v_sel = zvec
                for v in range(16):
                    cond = (abv[v] < req) & (abv[v] + tv[v] >= req)
                    s_sel = jnp.where(cond, shr_vm[pl.ds(v * 16, 16)], s_sel)
                    above_sel = jnp.where(cond, abv[v], above_sel)
                    v_sel = jnp.where(cond, jnp.full((16,), v, jnp.int32),
                                      v_sel)
                tsel = splat_sum(s_sel)
                prefx = csum(s_sel)
                suffix = tsel - prefx + s_sel
                ge = ((above_sel + suffix) >= req).astype(jnp.int32)
                d = splat_sum(ge) - 1
                digit = v_sel * 16 + d
                cnt_gt = above_sel + splat_sum(
                    s_sel * (lane > d).astype(jnp.int32))
                req = req - cnt_gt
                pref = pref * 256 + digit - (128 if t == 0 else 0)
            kth_vec = kth_vec + pref * (lane == bg).astype(jnp.int32)

        @pl.when(sid == 0)
        def _out():
            kth_vm[...] = kth_vec
            pltpu.sync_copy(kth_vm, out_hbm.at[core])

    return sel(logits_flat)


def kernel(z_s, env_idx, A_base, A_deltas, temperature):
    del z_s
    b, d = _B, _D
    env = env_idx.astype(jnp.int32)
    temp = jnp.asarray(temperature, jnp.float32).reshape(1)

    build_spec = pltpu.PrefetchScalarGridSpec(
        num_scalar_prefetch=1,
        grid=(b, _C),
        in_specs=[
            pl.BlockSpec(memory_space=pltpu.MemorySpace.SMEM),
            pl.BlockSpec(
                (_CHUNK, d),
                lambda bi, i, e: (jnp.where(bi == 0, i, _C - 1), 0)),
            pl.BlockSpec((1, _CHUNK, d), lambda bi, i, e: (e[bi], i, 0)),
        ],
        out_specs=[
            pl.BlockSpec((1, _CHUNK, d), lambda bi, i, e: (bi, i, 0)),
            pl.BlockSpec((1, _CHUNK, d), lambda bi, i, e: (bi, i, 0)),
        ],
        scratch_shapes=[
            pltpu.MemorySpace.VMEM((d, d), jnp.float32),
        ],
    )
    logits, soft = pl.pallas_call(
        _build_body,
        grid_spec=build_spec,
        out_shape=[
            jax.ShapeDtypeStruct((b, d, d), jnp.float32),
            jax.ShapeDtypeStruct((b, d, d), jnp.float32),
        ],
    )(env, temp, A_base, A_deltas)

    kth2 = _sc_select(logits.reshape(b, d * d))
    kth16 = kth2.sum(axis=0)

    emit_spec = pltpu.PrefetchScalarGridSpec(
        num_scalar_prefetch=1,
        grid=(b, _C),
        in_specs=[
            pl.BlockSpec((1, _CHUNK, d), lambda bi, i, e: (bi, i, 0)),
        ],
        out_specs=[
            pl.BlockSpec((1, _CHUNK, d), lambda bi, i, e: (bi, i, 0)),
        ],
    )
    a = pl.pallas_call(
        _emit_body,
        grid_spec=emit_spec,
        out_shape=[jax.ShapeDtypeStruct((b, d, d), jnp.float32)],
    )(kth16, logits)[0]
    return (a, logits, soft)


# speedup vs baseline: 1.0195x; 1.0195x over previous
import functools

import numpy as np

import jax
import jax.numpy as jnp
from jax.experimental import pallas as pl
from jax.experimental.pallas import tpu as pltpu
from jax.experimental.pallas import tpu_sc as plsc

_D = 1024
_B = 8
_K = max(1, int(0.1 * _D * _D))
_CHUNK = 128
_C = _D // _CHUNK
_MASK31 = 0x7FFFFFFF
_TPB = (_D * _D) // 16
_HALF = _TPB // 2


def _to_key(x):
    bits = jax.lax.bitcast_convert_type(x, jnp.int32)
    return jnp.where(bits < 0, bits ^ _MASK31, bits)


def _build_body(env_ref, temp_ref, base_ref, delta_ref,
                logits_ref, soft_ref, base_vmem):
    b = pl.program_id(0)
    i = pl.program_id(1)
    row = i * _CHUNK

    @pl.when(b == 0)
    def _fill_cache():
        base_vmem[pl.ds(row, _CHUNK), :] = base_ref[...]

    x = base_vmem[pl.ds(row, _CHUNK), :] + delta_ref[0]
    logits_ref[0] = x
    soft_ref[0] = jax.nn.sigmoid(x * (1.0 / temp_ref[0]))


def _emit_body(kth_ref, logits_ref, a_ref):
    b = pl.program_id(0)
    x = logits_ref[0]
    key = _to_key(x)
    kth = kth_ref[b]
    a_ref[0] = jnp.where(key >= kth, jax.nn.sigmoid(x), 0.0)


def _sc_select(logits_flat):
    mesh = plsc.VectorSubcoreMesh(core_axis_name="c", subcore_axis_name="s")

    @functools.partial(
        pl.kernel, mesh=mesh,
        compiler_params=pltpu.CompilerParams(needs_layout_passes=False),
        out_type=jax.ShapeDtypeStruct((2, 16), jnp.int32),
        scratch_types=[
            pltpu.VMEM((_TPB,), jnp.float32),
            pltpu.VMEM((4096,), jnp.int32),
            pltpu.VMEM((256,), jnp.int32),
            pltpu.VMEM((256,), jnp.int32),
            pltpu.VMEM((256,), jnp.int32),
            pltpu.VMEM((16,), jnp.int32),
            pltpu.VMEM((256,), jnp.int32),
            pltpu.VMEM_SHARED((256,), jnp.int32),
        ],
    )
    def sel(logits_hbm, out_hbm,
            keys_vm, hist_vm, red_vm, shr_vm, zeros_vm, kth_vm, iota_vm,
            shared):
        core = jax.lax.axis_index("c")
        sid = jax.lax.axis_index("s")
        lane = jax.lax.iota(jnp.int32, 16)
        zvec = jnp.zeros((16,), jnp.int32)
        ones = jnp.ones((16,), jnp.int32)

        def _perm(v, idx):
            return jax.lax.gather(
                v, idx[:, None],
                jax.lax.GatherDimensionNumbers(
                    offset_dims=(), collapsed_slice_dims=(0,),
                    start_index_map=(0,)),
                (1,), mode=jax.lax.GatherScatterMode.PROMISE_IN_BOUNDS)

        def splat_sum(v):
            for kk in (1, 2, 4, 8):
                v = v + _perm(v, lane ^ kk)
            return v

        def csum(v):
            for kk in (1, 2, 4, 8):
                g = _perm(v, jnp.maximum(lane - kk, 0))
                v = v + jnp.where(lane >= kk, g, zvec)
            return v

        for v in range(16):
            zeros_vm[pl.ds(v * 16, 16)] = zvec
            iota_vm[pl.ds(v * 16, 16)] = lane + v * 16
        kth_vec = zvec
        off = sid * _TPB
        un = 8
        for b in range(4):
            bg = core * 4 + b
            pltpu.sync_copy(logits_hbm.at[bg, pl.ds(off, _TPB)], keys_vm)
            req = jnp.full((16,), _K, jnp.int32)
            pref = zvec
            for t in range(4):
                def _zh(v, _):
                    hist_vm[pl.ds(v * 16, 16)] = zvec
                    return 0

                jax.lax.fori_loop(0, 256, _zh, 0)
                shd = 24 - 8 * t

                def _scan(g, _, t=t, shd=shd, pref=pref):
                    for u in range(un):
                        ptr = (g * un + u) * 16
                        raw = keys_vm[pl.ds(ptr, 16)]
                        if t == 0:
                            key = _to_key(raw)
                            keys_vm[pl.ds(ptr, 16)] = (
                                jax.lax.bitcast_convert_type(key, jnp.float32))
                            dig = jax.lax.shift_right_arithmetic(key, 24) + 128
                            idx = dig * 16 + lane
                            plsc.addupdate_scatter(hist_vm, [idx], ones,
                                                   mask=lane >= 0)
                        else:
                            kv = jax.lax.bitcast_convert_type(raw, jnp.int32)
                            dig = jax.lax.shift_right_arithmetic(kv, shd) & 255
                            hi = jax.lax.shift_right_arithmetic(kv, shd + 8)
                            idx = dig * 16 + lane
                            plsc.addupdate_scatter(hist_vm, [idx], ones,
                                                   mask=hi == pref)
                    return 0

                jax.lax.fori_loop(0, _TPB // 16 // un, _scan, 0)

                def _red(blk, _):
                    acc = zvec
                    for j in range(16):
                        tot = splat_sum(
                            hist_vm[pl.ds(blk * 256 + j * 16, 16)])
                        acc = acc + tot * (lane == j).astype(jnp.int32)
                    red_vm[pl.ds(blk * 16, 16)] = acc
                    return 0

                jax.lax.fori_loop(0, 16, _red, 0)
                plsc.subcore_barrier()

                @pl.when(sid == 0)
                def _z():
                    pltpu.sync_copy(zeros_vm, shared)

                plsc.subcore_barrier()
                pltpu.sync_copy(red_vm, shared.at[iota_vm], add=True)
                plsc.subcore_barrier()
                pltpu.sync_copy(shared, shr_vm)
                tv = [splat_sum(shr_vm[pl.ds(v * 16, 16)]) for v in range(16)]
                abv = [None] * 16
                acc = zvec
                for v in range(15, -1, -1):
                    abv[v] = acc
                    acc = acc + tv[v]
                s_sel = zvec
                above_sel = zvec
                v_sel = zvec
                for v in range(16):
                    cond = (abv[v] < req) & (abv[v] + tv[v] >= req)
                    s_sel = jnp.where(cond, shr_vm[pl.ds(v * 16, 16)], s_sel)
                    above_sel = jnp.where(cond, abv[v], above_sel)
                    v_sel = jnp.where(cond, jnp.full((16,), v, jnp.int32),
                                      v_sel)
                tsel = splat_sum(s_sel)
                prefx = csum(s_sel)
                suffix = tsel - prefx + s_sel
                ge = ((above_sel + suffix) >= req).astype(jnp.int32)
                d = splat_sum(ge) - 1
                digit = v_sel * 16 + d
                cnt_gt = above_sel + splat_sum(
                    s_sel * (lane > d).astype(jnp.int32))
                req = req - cnt_gt
                pref = pref * 256 + digit - (128 if t == 0 else 0)
            kth_vec = kth_vec + pref * (lane == bg).astype(jnp.int32)

        @pl.when(sid == 0)
        def _out():
            kth_vm[...] = kth_vec
            pltpu.sync_copy(kth_vm, out_hbm.at[core])

    return sel(logits_flat)


def kernel(z_s, env_idx, A_base, A_deltas, temperature):
    del z_s
    b, d = _B, _D
    env = env_idx.astype(jnp.int32)
    temp = jnp.asarray(temperature, jnp.float32).reshape(1)

    build_spec = pltpu.PrefetchScalarGridSpec(
        num_scalar_prefetch=1,
        grid=(b, _C),
        in_specs=[
            pl.BlockSpec(memory_space=pltpu.MemorySpace.SMEM),
            pl.BlockSpec(
                (_CHUNK, d),
                lambda bi, i, e: (jnp.where(bi == 0, i, _C - 1), 0)),
            pl.BlockSpec((1, _CHUNK, d), lambda bi, i, e: (e[bi], i, 0)),
        ],
        out_specs=[
            pl.BlockSpec((1, _CHUNK, d), lambda bi, i, e: (bi, i, 0)),
            pl.BlockSpec((1, _CHUNK, d), lambda bi, i, e: (bi, i, 0)),
        ],
        scratch_shapes=[
            pltpu.MemorySpace.VMEM((d, d), jnp.float32),
        ],
    )
    logits, soft = pl.pallas_call(
        _build_body,
        grid_spec=build_spec,
        out_shape=[
            jax.ShapeDtypeStruct((b, d, d), jnp.float32),
            jax.ShapeDtypeStruct((b, d, d), jnp.float32),
        ],
    )(env, temp, A_base, A_deltas)

    kth2 = _sc_select(logits.reshape(b, d * d))
    kth16 = kth2.sum(axis=0)

    emit_spec = pltpu.PrefetchScalarGridSpec(
        num_scalar_prefetch=1,
        grid=(b, _C),
        in_specs=[
            pl.BlockSpec((1, _CHUNK, d), lambda bi, i, e: (bi, i, 0)),
        ],
        out_specs=[
            pl.BlockSpec((1, _CHUNK, d), lambda bi, i, e: (bi, i, 0)),
        ],
    )
    a = pl.pallas_call(
        _emit_body,
        grid_spec=emit_spec,
        out_shape=[jax.ShapeDtypeStruct((b, d, d), jnp.float32)],
    )(kth16, logits)[0]
    return (a, logits, soft)


# speedup vs baseline: 3.7113x; 3.6403x over previous
import numpy as np

import jax
import jax.numpy as jnp
from jax.experimental import pallas as pl
from jax.experimental.pallas import tpu as pltpu

_D = 1024
_B = 8
_K = max(1, int(0.1 * _D * _D))
_CHUNK = 128
_C = _D // _CHUNK
_LPS = 4
_P = _B + 2
_MASK31 = 0x7FFFFFFF


def _to_key(x):
    bits = jax.lax.bitcast_convert_type(x, jnp.int32)
    return jnp.where(bits < 0, bits ^ _MASK31, bits)


def _from_key(key):
    bits = jnp.where(key < 0, key ^ _MASK31, key)
    return jax.lax.bitcast_convert_type(bits, jnp.float32)


def _body(env_ref, temp_ref, base_ref, delta_ref,
          a_ref, logits_ref, soft_ref, keys3_ref, base_vmem,
          state_ref, kth_ref):
    p = pl.program_id(0)
    i = pl.program_id(1)

    @pl.when(p < _B)
    def _build():
        row = i * _CHUNK

        @pl.when(p == 0)
        def _fill_cache():
            base_vmem[pl.ds(row, _CHUNK), :] = base_ref[...]

        x = base_vmem[pl.ds(row, _CHUNK), :] + delta_ref[0]
        logits_ref[0] = x
        soft_ref[0] = jax.nn.sigmoid(x * (1.0 / temp_ref[0]))
        bufp = jax.lax.rem(p, 3)
        keys3_ref[bufp, pl.ds(row, _CHUNK), :] = _to_key(x)

    @pl.when((p >= 1) & (p <= _B))
    def _count():
        bufq = jax.lax.rem(p - 1, 3)
        kb = keys3_ref.at[bufq]

        @pl.when(i == 0)
        def _init():
            state_ref[0] = np.int32(-2147483648)

        lo = state_ref[0]
        kk = np.int32(_K)
        nslc = 8
        rows = _D // nslc
        slices = [kb[j * rows:(j + 1) * rows, :] for j in range(nslc)]
        base_level = i * _LPS
        for l in range(_LPS):
            shift = 31 - (base_level + l)
            delta = np.int32(1) << shift
            mid = lo + delta
            parts = [jnp.sum((sl >= mid).astype(jnp.int32)) for sl in slices]
            cnt = sum(parts)
            lo = jnp.where(cnt >= kk, mid, lo)
        state_ref[0] = lo
        kth_ref[jnp.clip(p - 1, 0, _B - 1)] = lo

    @pl.when(p >= 2)
    def _emit():
        bufr = jax.lax.rem(p - 2, 3)
        kth = kth_ref[jnp.clip(p - 2, 0, _B - 1)]
        key = keys3_ref[bufr, pl.ds(i * _CHUNK, _CHUNK), :]
        x = _from_key(key)
        a_ref[0] = jnp.where(key >= kth, jax.nn.sigmoid(x), 0.0)


def kernel(z_s, env_idx, A_base, A_deltas, temperature):
    del z_s
    b, d = _B, _D
    env = env_idx.astype(jnp.int32)
    temp = jnp.asarray(temperature, jnp.float32).reshape(1)

    grid_spec = pltpu.PrefetchScalarGridSpec(
        num_scalar_prefetch=1,
        grid=(_P, _C),
        in_specs=[
            pl.BlockSpec(memory_space=pltpu.MemorySpace.SMEM),
            pl.BlockSpec(
                (_CHUNK, d),
                lambda p, i, e: (jnp.where(p == 0, i, _C - 1), 0)),
            pl.BlockSpec(
                (1, _CHUNK, d),
                lambda p, i, e: (e[jnp.clip(p, 0, _B - 1)],
                                 jnp.where(p < _B, i, _C - 1), 0)),
        ],
        out_specs=[
            pl.BlockSpec(
                (1, _CHUNK, d),
                lambda p, i, e: (jnp.clip(p - 2, 0, _B - 1),
                                 jnp.where(p >= 2, i, 0), 0)),
            pl.BlockSpec(
                (1, _CHUNK, d),
                lambda p, i, e: (jnp.clip(p, 0, _B - 1),
                                 jnp.where(p < _B, i, _C - 1), 0)),
            pl.BlockSpec(
                (1, _CHUNK, d),
                lambda p, i, e: (jnp.clip(p, 0, _B - 1),
                                 jnp.where(p < _B, i, _C - 1), 0)),
        ],
        scratch_shapes=[
            pltpu.MemorySpace.VMEM((3, d, d), jnp.int32),
            pltpu.MemorySpace.VMEM((d, d), jnp.float32),
            pltpu.MemorySpace.SMEM((1,), jnp.int32),
            pltpu.MemorySpace.SMEM((_B,), jnp.int32),
        ],
    )
    out_shape = [
        jax.ShapeDtypeStruct((b, d, d), jnp.float32),
        jax.ShapeDtypeStruct((b, d, d), jnp.float32),
        jax.ShapeDtypeStruct((b, d, d), jnp.float32),
    ]
    a, logits, soft = pl.pallas_call(
        _body,
        grid_spec=grid_spec,
        out_shape=out_shape,
    )(env, temp, A_base, A_deltas)
    return (a, logits, soft)


# speedup vs baseline: 3.9922x; 1.0757x over previous
import numpy as np

import jax
import jax.numpy as jnp
from jax.experimental import pallas as pl
from jax.experimental.pallas import tpu as pltpu

_D = 1024
_B = 8
_K = max(1, int(0.1 * _D * _D))
_CHUNK = 128
_C = _D // _CHUNK
_LPS = 4
_P = _B + 2
_MASK31 = 0x7FFFFFFF


def _to_key(x):
    bits = jax.lax.bitcast_convert_type(x, jnp.int32)
    return jnp.where(bits < 0, bits ^ _MASK31, bits)


def _from_key(key):
    bits = jnp.where(key < 0, key ^ _MASK31, key)
    return jax.lax.bitcast_convert_type(bits, jnp.float32)


def _body(env_ref, temp_ref, base_ref, delta_ref,
          a_ref, logits_ref, soft_ref, keys3_ref, base_vmem,
          state_ref, kth_ref):
    p = pl.program_id(0)
    i = pl.program_id(1)

    @pl.when(p < _B)
    def _build():
        row = i * _CHUNK

        @pl.when(p == 0)
        def _fill_cache():
            base_vmem[pl.ds(row, _CHUNK), :] = base_ref[...]

        x = base_vmem[pl.ds(row, _CHUNK), :] + delta_ref[0]
        logits_ref[0] = x
        soft_ref[0] = jax.nn.sigmoid(x * (1.0 / temp_ref[0]))
        bufp = jax.lax.rem(p, 3)
        keys3_ref[bufp, pl.ds(row, _CHUNK), :] = _to_key(x)

    @pl.when((p >= 1) & (p <= _B))
    def _count():
        bufq = jax.lax.rem(p - 1, 3)
        kb = keys3_ref.at[bufq]

        @pl.when(i == 0)
        def _init():
            state_ref[0] = np.int32(-2147483648)

        lo = state_ref[0]
        kk = np.int32(_K)
        nslc = 16
        rows = _D // nslc
        slices = [kb[j * rows:(j + 1) * rows, :] for j in range(nslc)]
        base_level = i * _LPS
        for l in range(_LPS):
            shift = 31 - (base_level + l)
            delta = np.int32(1) << shift
            mid = lo + delta
            parts = [jnp.sum((sl >= mid).astype(jnp.int32)) for sl in slices]
            cnt = sum(parts)
            lo = jnp.where(cnt >= kk, mid, lo)
        state_ref[0] = lo
        kth_ref[jnp.clip(p - 1, 0, _B - 1)] = lo

    @pl.when(p >= 2)
    def _emit():
        bufr = jax.lax.rem(p - 2, 3)
        kth = kth_ref[jnp.clip(p - 2, 0, _B - 1)]
        key = keys3_ref[bufr, pl.ds(i * _CHUNK, _CHUNK), :]
        x = _from_key(key)
        a_ref[0] = jnp.where(key >= kth, jax.nn.sigmoid(x), 0.0)


def kernel(z_s, env_idx, A_base, A_deltas, temperature):
    del z_s
    b, d = _B, _D
    env = env_idx.astype(jnp.int32)
    temp = jnp.asarray(temperature, jnp.float32).reshape(1)

    grid_spec = pltpu.PrefetchScalarGridSpec(
        num_scalar_prefetch=1,
        grid=(_P, _C),
        in_specs=[
            pl.BlockSpec(memory_space=pltpu.MemorySpace.SMEM),
            pl.BlockSpec(
                (_CHUNK, d),
                lambda p, i, e: (jnp.where(p == 0, i, _C - 1), 0)),
            pl.BlockSpec(
                (1, _CHUNK, d),
                lambda p, i, e: (e[jnp.clip(p, 0, _B - 1)],
                                 jnp.where(p < _B, i, _C - 1), 0)),
        ],
        out_specs=[
            pl.BlockSpec(
                (1, _CHUNK, d),
                lambda p, i, e: (jnp.clip(p - 2, 0, _B - 1),
                                 jnp.where(p >= 2, i, 0), 0)),
            pl.BlockSpec(
                (1, _CHUNK, d),
                lambda p, i, e: (jnp.clip(p, 0, _B - 1),
                                 jnp.where(p < _B, i, _C - 1), 0)),
            pl.BlockSpec(
                (1, _CHUNK, d),
                lambda p, i, e: (jnp.clip(p, 0, _B - 1),
                                 jnp.where(p < _B, i, _C - 1), 0)),
        ],
        scratch_shapes=[
            pltpu.MemorySpace.VMEM((3, d, d), jnp.int32),
            pltpu.MemorySpace.VMEM((d, d), jnp.float32),
            pltpu.MemorySpace.SMEM((1,), jnp.int32),
            pltpu.MemorySpace.SMEM((_B,), jnp.int32),
        ],
    )
    out_shape = [
        jax.ShapeDtypeStruct((b, d, d), jnp.float32),
        jax.ShapeDtypeStruct((b, d, d), jnp.float32),
        jax.ShapeDtypeStruct((b, d, d), jnp.float32),
    ]
    a, logits, soft = pl.pallas_call(
        _body,
        grid_spec=grid_spec,
        out_shape=out_shape,
    )(env, temp, A_base, A_deltas)
    return (a, logits, soft)
